# baseline (speedup 1.0000x reference)
import jax
import jax.numpy as jnp
from jax.experimental import pallas as pl
from jax.experimental.pallas import tpu as pltpu

_CHUNK = 16


def _split_bf16(a):
    hi = a.astype(jnp.bfloat16)
    lo = (a - hi.astype(jnp.float32)).astype(jnp.bfloat16)
    return hi, lo


def _mm3(a_f32, bh, bl):
    rows = a_f32.shape[0]
    ah, al = _split_bf16(a_f32)
    a2 = jnp.concatenate([ah, al], axis=0)
    p = jnp.dot(a2, bh, preferred_element_type=jnp.float32)
    q = jnp.dot(ah, bl, preferred_element_type=jnp.float32)
    return p[:rows] + p[rows:] + q


def _gru_steps(chunk, nb, hdim, h, m_ref, gi_scr, whh_h_ref, whh_l_ref,
               bhhn_ref, out_wr):
    for i in range(chunk):
        m_t = m_ref[i]
        hm = h * m_t
        gh = _mm3(hm, whh_h_ref[...], whh_l_ref[...])
        gi_t = gi_scr[i * nb:(i + 1) * nb, :]
        r = jax.nn.sigmoid(gi_t[:, :hdim] + gh[:, :hdim])
        z = jax.nn.sigmoid(gi_t[:, hdim:2 * hdim] + gh[:, hdim:2 * hdim])
        n = jnp.tanh(gi_t[:, 2 * hdim:]
                     + r * (gh[:, 2 * hdim:] + bhhn_ref[...]))
        h = (1.0 - z) * n + z * hm
        out_wr(i, h)
    return h


def _fused_kernel(x_ref, m_ref, h0_ref, h1_ref,
                  wih0_h_ref, wih0_l_ref, bi0_ref,
                  whh0_h_ref, whh0_l_ref, bn0_ref,
                  wih1_h_ref, wih1_l_ref, bi1_ref,
                  whh1_h_ref, whh1_l_ref, bn1_ref,
                  out_ref, h0n_ref, h1n_ref,
                  h0_scr, h1_scr, gi_scr, out0_scr):
    c = pl.program_id(0)
    nchunks = pl.num_programs(0)
    hdim = h0_ref.shape[-1]
    chunk = m_ref.shape[0]
    nb = h0_ref.shape[0]

    @pl.when(c == 0)
    def _():
        h0_scr[...] = h0_ref[...]
        h1_scr[...] = h1_ref[...]

    gi_scr[...] = _mm3(x_ref[...], wih0_h_ref[...], wih0_l_ref[...]) \
        + bi0_ref[...]

    def wr0(i, h):
        out0_scr[i * nb:(i + 1) * nb, :] = h

    h0 = _gru_steps(chunk, nb, hdim, h0_scr[...], m_ref, gi_scr,
                    whh0_h_ref, whh0_l_ref, bn0_ref, wr0)
    h0_scr[...] = h0

    gi_scr[...] = _mm3(out0_scr[...], wih1_h_ref[...], wih1_l_ref[...]) \
        + bi1_ref[...]

    def wr1(i, h):
        out_ref[i * nb:(i + 1) * nb, :] = h

    h1 = _gru_steps(chunk, nb, hdim, h1_scr[...], m_ref, gi_scr,
                    whh1_h_ref, whh1_l_ref, bn1_ref, wr1)
    h1_scr[...] = h1

    @pl.when(c == nchunks - 1)
    def _():
        h0n_ref[...] = h0
        h1n_ref[...] = h1


def kernel(x, hidden_states, masks, W_ih_0, W_hh_0, b_ih_0, b_hh_0,
           W_ih_1, W_hh_1, b_ih_1, b_hh_1):
    n = hidden_states.shape[1]
    tn, d = x.shape
    t = tn // n
    h = hidden_states.shape[2]
    g3 = 3 * h
    chunk = _CHUNK
    rows = chunk * n

    m3 = masks.reshape(t, n, 1).astype(jnp.float32)

    wih0_h, wih0_l = _split_bf16(W_ih_0.T)
    whh0_h, whh0_l = _split_bf16(W_hh_0.T)
    wih1_h, wih1_l = _split_bf16(W_ih_1.T)
    whh1_h, whh1_l = _split_bf16(W_hh_1.T)

    def gi_bias(b_ih, b_hh):
        return jnp.concatenate(
            [b_ih[:2 * h] + b_hh[:2 * h], b_ih[2 * h:]]).reshape(1, g3)

    bi0 = gi_bias(b_ih_0, b_hh_0)
    bi1 = gi_bias(b_ih_1, b_hh_1)
    bn0 = b_hh_0[2 * h:].reshape(1, h)
    bn1 = b_hh_1[2 * h:].reshape(1, h)

    wspec = [
        pl.BlockSpec((d, g3), lambda c: (0, 0)),
        pl.BlockSpec((d, g3), lambda c: (0, 0)),
        pl.BlockSpec((1, g3), lambda c: (0, 0)),
        pl.BlockSpec((h, g3), lambda c: (0, 0)),
        pl.BlockSpec((h, g3), lambda c: (0, 0)),
        pl.BlockSpec((1, h), lambda c: (0, 0)),
        pl.BlockSpec((h, g3), lambda c: (0, 0)),
        pl.BlockSpec((h, g3), lambda c: (0, 0)),
        pl.BlockSpec((1, g3), lambda c: (0, 0)),
        pl.BlockSpec((h, g3), lambda c: (0, 0)),
        pl.BlockSpec((h, g3), lambda c: (0, 0)),
        pl.BlockSpec((1, h), lambda c: (0, 0)),
    ]

    out, h0n, h1n = pl.pallas_call(
        _fused_kernel,
        grid=(t // chunk,),
        in_specs=[
            pl.BlockSpec((rows, d), lambda c: (c, 0)),
            pl.BlockSpec((chunk, n, 1), lambda c: (c, 0, 0)),
            pl.BlockSpec((n, h), lambda c: (0, 0)),
            pl.BlockSpec((n, h), lambda c: (0, 0)),
        ] + wspec,
        out_specs=[
            pl.BlockSpec((rows, h), lambda c: (c, 0)),
            pl.BlockSpec((n, h), lambda c: (0, 0)),
            pl.BlockSpec((n, h), lambda c: (0, 0)),
        ],
        out_shape=[
            jax.ShapeDtypeStruct((tn, h), jnp.float32),
            jax.ShapeDtypeStruct((n, h), jnp.float32),
            jax.ShapeDtypeStruct((n, h), jnp.float32),
        ],
        scratch_shapes=[
            pltpu.VMEM((n, h), jnp.float32),
            pltpu.VMEM((n, h), jnp.float32),
            pltpu.VMEM((rows, g3), jnp.float32),
            pltpu.VMEM((rows, h), jnp.float32),
        ],
        compiler_params=pltpu.CompilerParams(
            dimension_semantics=("arbitrary",),
        ),
    )(x, m3, hidden_states[0], hidden_states[1],
      wih0_h, wih0_l, bi0, whh0_h, whh0_l, bn0,
      wih1_h, wih1_l, bi1, whh1_h, whh1_l, bn1)

    return out, jnp.stack([h0n, h1n], axis=0)


# speedup vs baseline: 1.8925x; 1.8925x over previous
import jax
import jax.numpy as jnp
from jax.experimental import pallas as pl
from jax.experimental.pallas import tpu as pltpu

_CHUNK = 16


def _gru_steps(chunk, nb, hdim, h, m_ref, gi_scr, whh_ref, bhhn_ref, out_wr):
    for i in range(chunk):
        m_t = m_ref[i]
        hm = h * m_t
        hm_b = hm.astype(jnp.bfloat16)
        gi_t = gi_scr[i * nb:(i + 1) * nb, :]
        gh_r = jnp.dot(hm_b, whh_ref[:, :hdim],
                       preferred_element_type=jnp.float32)
        gh_z = jnp.dot(hm_b, whh_ref[:, hdim:2 * hdim],
                       preferred_element_type=jnp.float32)
        gh_n = jnp.dot(hm_b, whh_ref[:, 2 * hdim:],
                       preferred_element_type=jnp.float32)
        r = jax.nn.sigmoid(gi_t[:, :hdim] + gh_r)
        z = jax.nn.sigmoid(gi_t[:, hdim:2 * hdim] + gh_z)
        n = jnp.tanh(gi_t[:, 2 * hdim:] + r * (gh_n + bhhn_ref[...]))
        h = (1.0 - z) * n + z * hm
        out_wr(i, h)
    return h


def _fused_kernel(x_ref, m_ref, h0_ref, h1_ref,
                  wih0_ref, bi0_ref, whh0_ref, bn0_ref,
                  wih1_ref, bi1_ref, whh1_ref, bn1_ref,
                  out_ref, h0n_ref, h1n_ref,
                  h0_scr, h1_scr, gi_scr, out0_scr):
    c = pl.program_id(0)
    nchunks = pl.num_programs(0)
    hdim = h0_ref.shape[-1]
    chunk = m_ref.shape[0]
    nb = h0_ref.shape[0]

    @pl.when(c == 0)
    def _():
        h0_scr[...] = h0_ref[...]
        h1_scr[...] = h1_ref[...]

    gi_scr[...] = jnp.dot(x_ref[...].astype(jnp.bfloat16), wih0_ref[...],
                          preferred_element_type=jnp.float32) + bi0_ref[...]

    def wr0(i, h):
        out0_scr[i * nb:(i + 1) * nb, :] = h.astype(jnp.bfloat16)

    h0 = _gru_steps(chunk, nb, hdim, h0_scr[...], m_ref, gi_scr,
                    whh0_ref, bn0_ref, wr0)
    h0_scr[...] = h0

    gi_scr[...] = jnp.dot(out0_scr[...], wih1_ref[...],
                          preferred_element_type=jnp.float32) + bi1_ref[...]

    def wr1(i, h):
        out_ref[i * nb:(i + 1) * nb, :] = h

    h1 = _gru_steps(chunk, nb, hdim, h1_scr[...], m_ref, gi_scr,
                    whh1_ref, bn1_ref, wr1)
    h1_scr[...] = h1

    @pl.when(c == nchunks - 1)
    def _():
        h0n_ref[...] = h0
        h1n_ref[...] = h1


def kernel(x, hidden_states, masks, W_ih_0, W_hh_0, b_ih_0, b_hh_0,
           W_ih_1, W_hh_1, b_ih_1, b_hh_1):
    n = hidden_states.shape[1]
    tn, d = x.shape
    t = tn // n
    h = hidden_states.shape[2]
    g3 = 3 * h
    chunk = _CHUNK
    rows = chunk * n

    m3 = masks.reshape(t, n, 1).astype(jnp.float32)

    def gi_bias(b_ih, b_hh):
        return jnp.concatenate(
            [b_ih[:2 * h] + b_hh[:2 * h], b_ih[2 * h:]]).reshape(1, g3)

    bi0 = gi_bias(b_ih_0, b_hh_0)
    bi1 = gi_bias(b_ih_1, b_hh_1)
    bn0 = b_hh_0[2 * h:].reshape(1, h)
    bn1 = b_hh_1[2 * h:].reshape(1, h)

    out, h0n, h1n = pl.pallas_call(
        _fused_kernel,
        grid=(t // chunk,),
        in_specs=[
            pl.BlockSpec((rows, d), lambda c: (c, 0)),
            pl.BlockSpec((chunk, n, 1), lambda c: (c, 0, 0)),
            pl.BlockSpec((n, h), lambda c: (0, 0)),
            pl.BlockSpec((n, h), lambda c: (0, 0)),
            pl.BlockSpec((d, g3), lambda c: (0, 0)),
            pl.BlockSpec((1, g3), lambda c: (0, 0)),
            pl.BlockSpec((h, g3), lambda c: (0, 0)),
            pl.BlockSpec((1, h), lambda c: (0, 0)),
            pl.BlockSpec((h, g3), lambda c: (0, 0)),
            pl.BlockSpec((1, g3), lambda c: (0, 0)),
            pl.BlockSpec((h, g3), lambda c: (0, 0)),
            pl.BlockSpec((1, h), lambda c: (0, 0)),
        ],
        out_specs=[
            pl.BlockSpec((rows, h), lambda c: (c, 0)),
            pl.BlockSpec((n, h), lambda c: (0, 0)),
            pl.BlockSpec((n, h), lambda c: (0, 0)),
        ],
        out_shape=[
            jax.ShapeDtypeStruct((tn, h), jnp.float32),
            jax.ShapeDtypeStruct((n, h), jnp.float32),
            jax.ShapeDtypeStruct((n, h), jnp.float32),
        ],
        scratch_shapes=[
            pltpu.VMEM((n, h), jnp.float32),
            pltpu.VMEM((n, h), jnp.float32),
            pltpu.VMEM((rows, g3), jnp.float32),
            pltpu.VMEM((rows, h), jnp.bfloat16),
        ],
        compiler_params=pltpu.CompilerParams(
            dimension_semantics=("arbitrary",),
        ),
    )(x, m3, hidden_states[0], hidden_states[1],
      W_ih_0.T.astype(jnp.bfloat16), bi0, W_hh_0.T.astype(jnp.bfloat16), bn0,
      W_ih_1.T.astype(jnp.bfloat16), bi1, W_hh_1.T.astype(jnp.bfloat16), bn1)

    return out, jnp.stack([h0n, h1n], axis=0)


# speedup vs baseline: 1.8937x; 1.0006x over previous
import jax
import jax.numpy as jnp
from jax.experimental import pallas as pl
from jax.experimental.pallas import tpu as pltpu

_CHUNK = 16


def _gru_steps(chunk, nb, hdim, h, m_ref, gi_scr, whh_ref, bhhn_ref, out_wr):
    for i in range(chunk):
        m_t = m_ref[i]
        hm = h * m_t
        hm_b = hm.astype(jnp.bfloat16)
        gi_t = gi_scr[i * nb:(i + 1) * nb, :]
        gh_r = jnp.dot(hm_b, whh_ref[:, :hdim],
                       preferred_element_type=jnp.float32)
        gh_z = jnp.dot(hm_b, whh_ref[:, hdim:2 * hdim],
                       preferred_element_type=jnp.float32)
        gh_n = jnp.dot(hm_b, whh_ref[:, 2 * hdim:],
                       preferred_element_type=jnp.float32)
        r = 0.5 * jnp.tanh(0.5 * (gi_t[:, :hdim] + gh_r)) + 0.5
        z = 0.5 * jnp.tanh(0.5 * (gi_t[:, hdim:2 * hdim] + gh_z)) + 0.5
        n = jnp.tanh(gi_t[:, 2 * hdim:] + r * (gh_n + bhhn_ref[...]))
        h = n + z * (hm - n)
        out_wr(i, h)
    return h


def _fused_kernel(x_ref, m_ref, h0_ref, h1_ref,
                  wih0_ref, bi0_ref, whh0_ref, bn0_ref,
                  wih1_ref, bi1_ref, whh1_ref, bn1_ref,
                  out_ref, h0n_ref, h1n_ref,
                  h0_scr, h1_scr, gi_scr, out0_scr):
    c = pl.program_id(0)
    nchunks = pl.num_programs(0)
    hdim = h0_ref.shape[-1]
    chunk = m_ref.shape[0]
    nb = h0_ref.shape[0]

    @pl.when(c == 0)
    def _():
        h0_scr[...] = h0_ref[...]
        h1_scr[...] = h1_ref[...]

    gi_scr[...] = jnp.dot(x_ref[...].astype(jnp.bfloat16), wih0_ref[...],
                          preferred_element_type=jnp.float32) + bi0_ref[...]

    def wr0(i, h):
        out0_scr[i * nb:(i + 1) * nb, :] = h.astype(jnp.bfloat16)

    h0 = _gru_steps(chunk, nb, hdim, h0_scr[...], m_ref, gi_scr,
                    whh0_ref, bn0_ref, wr0)
    h0_scr[...] = h0

    gi_scr[...] = jnp.dot(out0_scr[...], wih1_ref[...],
                          preferred_element_type=jnp.float32) + bi1_ref[...]

    def wr1(i, h):
        out_ref[i * nb:(i + 1) * nb, :] = h

    h1 = _gru_steps(chunk, nb, hdim, h1_scr[...], m_ref, gi_scr,
                    whh1_ref, bn1_ref, wr1)
    h1_scr[...] = h1

    @pl.when(c == nchunks - 1)
    def _():
        h0n_ref[...] = h0
        h1n_ref[...] = h1


def kernel(x, hidden_states, masks, W_ih_0, W_hh_0, b_ih_0, b_hh_0,
           W_ih_1, W_hh_1, b_ih_1, b_hh_1):
    n = hidden_states.shape[1]
    tn, d = x.shape
    t = tn // n
    h = hidden_states.shape[2]
    g3 = 3 * h
    chunk = _CHUNK
    rows = chunk * n

    m3 = masks.reshape(t, n, 1).astype(jnp.float32)

    def gi_bias(b_ih, b_hh):
        return jnp.concatenate(
            [b_ih[:2 * h] + b_hh[:2 * h], b_ih[2 * h:]]).reshape(1, g3)

    bi0 = gi_bias(b_ih_0, b_hh_0)
    bi1 = gi_bias(b_ih_1, b_hh_1)
    bn0 = b_hh_0[2 * h:].reshape(1, h)
    bn1 = b_hh_1[2 * h:].reshape(1, h)

    out, h0n, h1n = pl.pallas_call(
        _fused_kernel,
        grid=(t // chunk,),
        in_specs=[
            pl.BlockSpec((rows, d), lambda c: (c, 0)),
            pl.BlockSpec((chunk, n, 1), lambda c: (c, 0, 0)),
            pl.BlockSpec((n, h), lambda c: (0, 0)),
            pl.BlockSpec((n, h), lambda c: (0, 0)),
            pl.BlockSpec((d, g3), lambda c: (0, 0)),
            pl.BlockSpec((1, g3), lambda c: (0, 0)),
            pl.BlockSpec((h, g3), lambda c: (0, 0)),
            pl.BlockSpec((1, h), lambda c: (0, 0)),
            pl.BlockSpec((h, g3), lambda c: (0, 0)),
            pl.BlockSpec((1, g3), lambda c: (0, 0)),
            pl.BlockSpec((h, g3), lambda c: (0, 0)),
            pl.BlockSpec((1, h), lambda c: (0, 0)),
        ],
        out_specs=[
            pl.BlockSpec((rows, h), lambda c: (c, 0)),
            pl.BlockSpec((n, h), lambda c: (0, 0)),
            pl.BlockSpec((n, h), lambda c: (0, 0)),
        ],
        out_shape=[
            jax.ShapeDtypeStruct((tn, h), jnp.float32),
            jax.ShapeDtypeStruct((n, h), jnp.float32),
            jax.ShapeDtypeStruct((n, h), jnp.float32),
        ],
        scratch_shapes=[
            pltpu.VMEM((n, h), jnp.float32),
            pltpu.VMEM((n, h), jnp.float32),
            pltpu.VMEM((rows, g3), jnp.float32),
            pltpu.VMEM((rows, h), jnp.bfloat16),
        ],
        compiler_params=pltpu.CompilerParams(
            dimension_semantics=("arbitrary",),
        ),
    )(x, m3, hidden_states[0], hidden_states[1],
      W_ih_0.T.astype(jnp.bfloat16), bi0, W_hh_0.T.astype(jnp.bfloat16), bn0,
      W_ih_1.T.astype(jnp.bfloat16), bi1, W_hh_1.T.astype(jnp.bfloat16), bn1)

    return out, jnp.stack([h0n, h1n], axis=0)


# speedup vs baseline: 1.9075x; 1.0073x over previous
import jax
import jax.numpy as jnp
from jax.experimental import pallas as pl
from jax.experimental.pallas import tpu as pltpu

_CHUNK = 32


def _gru_steps(chunk, nb, hdim, h, m_ref, gi_scr, whh_ref, bhhn_ref, out_wr):
    for i in range(chunk):
        m_t = m_ref[i]
        hm = h * m_t
        hm_b = hm.astype(jnp.bfloat16)
        gi_t = gi_scr[i * nb:(i + 1) * nb, :]
        gh_r = jnp.dot(hm_b, whh_ref[:, :hdim],
                       preferred_element_type=jnp.float32)
        gh_z = jnp.dot(hm_b, whh_ref[:, hdim:2 * hdim],
                       preferred_element_type=jnp.float32)
        gh_n = jnp.dot(hm_b, whh_ref[:, 2 * hdim:],
                       preferred_element_type=jnp.float32)
        r = 0.5 * jnp.tanh(0.5 * (gi_t[:, :hdim] + gh_r)) + 0.5
        z = 0.5 * jnp.tanh(0.5 * (gi_t[:, hdim:2 * hdim] + gh_z)) + 0.5
        n = jnp.tanh(gi_t[:, 2 * hdim:] + r * (gh_n + bhhn_ref[...]))
        h = n + z * (hm - n)
        out_wr(i, h)
    return h


def _fused_kernel(x_ref, m_ref, h0_ref, h1_ref,
                  wih0_ref, bi0_ref, whh0_ref, bn0_ref,
                  wih1_ref, bi1_ref, whh1_ref, bn1_ref,
                  out_ref, h0n_ref, h1n_ref,
                  h0_scr, h1_scr, gi_scr, out0_scr):
    c = pl.program_id(0)
    nchunks = pl.num_programs(0)
    hdim = h0_ref.shape[-1]
    chunk = m_ref.shape[0]
    nb = h0_ref.shape[0]

    @pl.when(c == 0)
    def _():
        h0_scr[...] = h0_ref[...]
        h1_scr[...] = h1_ref[...]

    gi_scr[...] = jnp.dot(x_ref[...].astype(jnp.bfloat16), wih0_ref[...],
                          preferred_element_type=jnp.float32) + bi0_ref[...]

    def wr0(i, h):
        out0_scr[i * nb:(i + 1) * nb, :] = h.astype(jnp.bfloat16)

    h0 = _gru_steps(chunk, nb, hdim, h0_scr[...], m_ref, gi_scr,
                    whh0_ref, bn0_ref, wr0)
    h0_scr[...] = h0

    gi_scr[...] = jnp.dot(out0_scr[...], wih1_ref[...],
                          preferred_element_type=jnp.float32) + bi1_ref[...]

    def wr1(i, h):
        out_ref[i * nb:(i + 1) * nb, :] = h

    h1 = _gru_steps(chunk, nb, hdim, h1_scr[...], m_ref, gi_scr,
                    whh1_ref, bn1_ref, wr1)
    h1_scr[...] = h1

    @pl.when(c == nchunks - 1)
    def _():
        h0n_ref[...] = h0
        h1n_ref[...] = h1


def kernel(x, hidden_states, masks, W_ih_0, W_hh_0, b_ih_0, b_hh_0,
           W_ih_1, W_hh_1, b_ih_1, b_hh_1):
    n = hidden_states.shape[1]
    tn, d = x.shape
    t = tn // n
    h = hidden_states.shape[2]
    g3 = 3 * h
    chunk = _CHUNK
    rows = chunk * n

    m3 = masks.reshape(t, n, 1).astype(jnp.float32)

    def gi_bias(b_ih, b_hh):
        return jnp.concatenate(
            [b_ih[:2 * h] + b_hh[:2 * h], b_ih[2 * h:]]).reshape(1, g3)

    bi0 = gi_bias(b_ih_0, b_hh_0)
    bi1 = gi_bias(b_ih_1, b_hh_1)
    bn0 = b_hh_0[2 * h:].reshape(1, h)
    bn1 = b_hh_1[2 * h:].reshape(1, h)

    out, h0n, h1n = pl.pallas_call(
        _fused_kernel,
        grid=(t // chunk,),
        in_specs=[
            pl.BlockSpec((rows, d), lambda c: (c, 0)),
            pl.BlockSpec((chunk, n, 1), lambda c: (c, 0, 0)),
            pl.BlockSpec((n, h), lambda c: (0, 0)),
            pl.BlockSpec((n, h), lambda c: (0, 0)),
            pl.BlockSpec((d, g3), lambda c: (0, 0)),
            pl.BlockSpec((1, g3), lambda c: (0, 0)),
            pl.BlockSpec((h, g3), lambda c: (0, 0)),
            pl.BlockSpec((1, h), lambda c: (0, 0)),
            pl.BlockSpec((h, g3), lambda c: (0, 0)),
            pl.BlockSpec((1, g3), lambda c: (0, 0)),
            pl.BlockSpec((h, g3), lambda c: (0, 0)),
            pl.BlockSpec((1, h), lambda c: (0, 0)),
        ],
        out_specs=[
            pl.BlockSpec((rows, h), lambda c: (c, 0)),
            pl.BlockSpec((n, h), lambda c: (0, 0)),
            pl.BlockSpec((n, h), lambda c: (0, 0)),
        ],
        out_shape=[
            jax.ShapeDtypeStruct((tn, h), jnp.float32),
            jax.ShapeDtypeStruct((n, h), jnp.float32),
            jax.ShapeDtypeStruct((n, h), jnp.float32),
        ],
        scratch_shapes=[
            pltpu.VMEM((n, h), jnp.float32),
            pltpu.VMEM((n, h), jnp.float32),
            pltpu.VMEM((rows, g3), jnp.float32),
            pltpu.VMEM((rows, h), jnp.bfloat16),
        ],
        compiler_params=pltpu.CompilerParams(
            dimension_semantics=("arbitrary",),
        ),
    )(x, m3, hidden_states[0], hidden_states[1],
      W_ih_0.T.astype(jnp.bfloat16), bi0, W_hh_0.T.astype(jnp.bfloat16), bn0,
      W_ih_1.T.astype(jnp.bfloat16), bi1, W_hh_1.T.astype(jnp.bfloat16), bn1)

    return out, jnp.stack([h0n, h1n], axis=0)


# speedup vs baseline: 1.9107x; 1.0017x over previous
import jax
import jax.numpy as jnp
from jax.experimental import pallas as pl
from jax.experimental.pallas import tpu as pltpu

_CHUNK = 32


def _gru_steps(chunk, nb, hdim, h, m_ref, gi_scr, whhz_ref, whhr_ref,
               whhn_ref, bhhn_ref, out_wr):
    for i in range(chunk):
        m_t = m_ref[i]
        hm = h * m_t
        hm_b = hm.astype(jnp.bfloat16)
        gi_t = gi_scr[i * nb:(i + 1) * nb, :]
        gh_z = jnp.dot(hm_b, whhz_ref[...],
                       preferred_element_type=jnp.float32)
        gh_r = jnp.dot(hm_b, whhr_ref[...],
                       preferred_element_type=jnp.float32)
        gh_n = jnp.dot(hm_b, whhn_ref[...],
                       preferred_element_type=jnp.float32)
        z = 0.5 * jnp.tanh(0.5 * (gi_t[:, hdim:2 * hdim] + gh_z)) + 0.5
        r = 0.5 * jnp.tanh(0.5 * (gi_t[:, :hdim] + gh_r)) + 0.5
        zhm = z * hm
        omz = 1.0 - z
        n = jnp.tanh(gi_t[:, 2 * hdim:] + r * (gh_n + bhhn_ref[...]))
        h = n * omz + zhm
        out_wr(i, h)
    return h


def _fused_kernel(x_ref, m_ref, h0_ref, h1_ref,
                  wih0_ref, bi0_ref, whh0z_ref, whh0r_ref, whh0n_ref, bn0_ref,
                  wih1_ref, bi1_ref, whh1z_ref, whh1r_ref, whh1n_ref, bn1_ref,
                  out_ref, h0n_ref, h1n_ref,
                  h0_scr, h1_scr, gi_scr, out0_scr):
    c = pl.program_id(0)
    nchunks = pl.num_programs(0)
    hdim = h0_ref.shape[-1]
    chunk = m_ref.shape[0]
    nb = h0_ref.shape[0]

    @pl.when(c == 0)
    def _():
        h0_scr[...] = h0_ref[...]
        h1_scr[...] = h1_ref[...]

    gi_scr[...] = jnp.dot(x_ref[...].astype(jnp.bfloat16), wih0_ref[...],
                          preferred_element_type=jnp.float32) + bi0_ref[...]

    def wr0(i, h):
        out0_scr[i * nb:(i + 1) * nb, :] = h.astype(jnp.bfloat16)

    h0 = _gru_steps(chunk, nb, hdim, h0_scr[...], m_ref, gi_scr,
                    whh0z_ref, whh0r_ref, whh0n_ref, bn0_ref, wr0)
    h0_scr[...] = h0

    gi_scr[...] = jnp.dot(out0_scr[...], wih1_ref[...],
                          preferred_element_type=jnp.float32) + bi1_ref[...]

    def wr1(i, h):
        out_ref[i * nb:(i + 1) * nb, :] = h

    h1 = _gru_steps(chunk, nb, hdim, h1_scr[...], m_ref, gi_scr,
                    whh1z_ref, whh1r_ref, whh1n_ref, bn1_ref, wr1)
    h1_scr[...] = h1

    @pl.when(c == nchunks - 1)
    def _():
        h0n_ref[...] = h0
        h1n_ref[...] = h1


def kernel(x, hidden_states, masks, W_ih_0, W_hh_0, b_ih_0, b_hh_0,
           W_ih_1, W_hh_1, b_ih_1, b_hh_1):
    n = hidden_states.shape[1]
    tn, d = x.shape
    t = tn // n
    h = hidden_states.shape[2]
    g3 = 3 * h
    chunk = _CHUNK
    rows = chunk * n

    m3 = masks.reshape(t, n, 1).astype(jnp.float32)

    def gi_bias(b_ih, b_hh):
        return jnp.concatenate(
            [b_ih[:2 * h] + b_hh[:2 * h], b_ih[2 * h:]]).reshape(1, g3)

    bi0 = gi_bias(b_ih_0, b_hh_0)
    bi1 = gi_bias(b_ih_1, b_hh_1)
    bn0 = b_hh_0[2 * h:].reshape(1, h)
    bn1 = b_hh_1[2 * h:].reshape(1, h)

    call = pl.pallas_call(
        _fused_kernel,
        grid=(t // chunk,),
        in_specs=[
            pl.BlockSpec((rows, d), lambda c: (c, 0)),
            pl.BlockSpec((chunk, n, 1), lambda c: (c, 0, 0)),
            pl.BlockSpec((n, h), lambda c: (0, 0)),
            pl.BlockSpec((n, h), lambda c: (0, 0)),
            pl.BlockSpec((d, g3), lambda c: (0, 0)),
            pl.BlockSpec((1, g3), lambda c: (0, 0)),
            pl.BlockSpec((h, h), lambda c: (0, 0)),
            pl.BlockSpec((h, h), lambda c: (0, 0)),
            pl.BlockSpec((h, h), lambda c: (0, 0)),
            pl.BlockSpec((1, h), lambda c: (0, 0)),
            pl.BlockSpec((h, g3), lambda c: (0, 0)),
            pl.BlockSpec((1, g3), lambda c: (0, 0)),
            pl.BlockSpec((h, h), lambda c: (0, 0)),
            pl.BlockSpec((h, h), lambda c: (0, 0)),
            pl.BlockSpec((h, h), lambda c: (0, 0)),
            pl.BlockSpec((1, h), lambda c: (0, 0)),
        ],
        out_specs=[
            pl.BlockSpec((rows, h), lambda c: (c, 0)),
            pl.BlockSpec((n, h), lambda c: (0, 0)),
            pl.BlockSpec((n, h), lambda c: (0, 0)),
        ],
        out_shape=[
            jax.ShapeDtypeStruct((tn, h), jnp.float32),
            jax.ShapeDtypeStruct((n, h), jnp.float32),
            jax.ShapeDtypeStruct((n, h), jnp.float32),
        ],
        scratch_shapes=[
            pltpu.VMEM((n, h), jnp.float32),
            pltpu.VMEM((n, h), jnp.float32),
            pltpu.VMEM((rows, g3), jnp.float32),
            pltpu.VMEM((rows, h), jnp.bfloat16),
        ],
        compiler_params=pltpu.CompilerParams(
            dimension_semantics=("arbitrary",),
        ),
    )

    whh0 = W_hh_0.T.astype(jnp.bfloat16)
    whh1 = W_hh_1.T.astype(jnp.bfloat16)
    args = (x, m3, hidden_states[0], hidden_states[1],
            W_ih_0.T.astype(jnp.bfloat16), bi0,
            whh0[:, h:2 * h], whh0[:, :h], whh0[:, 2 * h:], bn0,
            W_ih_1.T.astype(jnp.bfloat16), bi1,
            whh1[:, h:2 * h], whh1[:, :h], whh1[:, 2 * h:], bn1)

    out, h0n, h1n = call(*args)
    return out, jnp.stack([h0n, h1n], axis=0)


# speedup vs baseline: 2.3088x; 1.2083x over previous
import jax
import jax.numpy as jnp
from jax.experimental import pallas as pl
from jax.experimental.pallas import tpu as pltpu

_CHUNK = 32
_SUB = 8


def _gru_step(i, nb, hdim, h, m_ref, gi_scr, whh_ref, bhhn_ref):
    m_t = m_ref[i]
    hm = h * m_t
    hm_b = hm.astype(jnp.bfloat16)
    gi_t = gi_scr[i * nb:(i + 1) * nb, :]
    gh = jnp.dot(hm_b, whh_ref[...], preferred_element_type=jnp.float32)
    z = 0.5 * jnp.tanh(0.5 * (gi_t[:, hdim:2 * hdim]
                              + gh[:, hdim:2 * hdim])) + 0.5
    r = 0.5 * jnp.tanh(0.5 * (gi_t[:, :hdim] + gh[:, :hdim])) + 0.5
    n = jnp.tanh(gi_t[:, 2 * hdim:]
                 + r * (gh[:, 2 * hdim:] + bhhn_ref[...]))
    return n * (1.0 - z) + z * hm


def _fused_kernel(x_ref, m_ref, h0_ref, h1_ref,
                  wih0_ref, bi0_ref, whh0_ref, bn0_ref,
                  wih1_ref, bi1_ref, whh1_ref, bn1_ref,
                  out_ref, h0n_ref, h1n_ref,
                  h0_scr, h1_scr, gi0_scr, gi1_scr, out0_scr):
    c = pl.program_id(0)
    nchunks = pl.num_programs(0)
    hdim = h0_ref.shape[-1]
    chunk = m_ref.shape[0]
    nb = h0_ref.shape[0]
    sub = _SUB
    nsub = chunk // sub

    @pl.when(c == 0)
    def _():
        h0_scr[...] = h0_ref[...]
        h1_scr[...] = h1_ref[...]

    gi0_scr[...] = jnp.dot(x_ref[...].astype(jnp.bfloat16), wih0_ref[...],
                           preferred_element_type=jnp.float32) + bi0_ref[...]

    a = h0_scr[...]
    b = h1_scr[...]

    for k in range(nsub + 1):
        for i in range(sub):
            if k < nsub:
                i0 = k * sub + i
                a = _gru_step(i0, nb, hdim, a, m_ref, gi0_scr,
                              whh0_ref, bn0_ref)
                out0_scr[i0 * nb:(i0 + 1) * nb, :] = a.astype(jnp.bfloat16)
            if k >= 1:
                i1 = (k - 1) * sub + i
                b = _gru_step(i1, nb, hdim, b, m_ref, gi1_scr,
                              whh1_ref, bn1_ref)
                out_ref[i1 * nb:(i1 + 1) * nb, :] = b
        if k < nsub:
            r0, r1 = k * sub * nb, (k + 1) * sub * nb
            gi1_scr[r0:r1, :] = jnp.dot(
                out0_scr[r0:r1, :], wih1_ref[...],
                preferred_element_type=jnp.float32) + bi1_ref[...]

    h0_scr[...] = a
    h1_scr[...] = b

    @pl.when(c == nchunks - 1)
    def _():
        h0n_ref[...] = a
        h1n_ref[...] = b


def kernel(x, hidden_states, masks, W_ih_0, W_hh_0, b_ih_0, b_hh_0,
           W_ih_1, W_hh_1, b_ih_1, b_hh_1):
    n = hidden_states.shape[1]
    tn, d = x.shape
    t = tn // n
    h = hidden_states.shape[2]
    g3 = 3 * h
    chunk = _CHUNK
    rows = chunk * n

    m3 = masks.reshape(t, n, 1).astype(jnp.float32)

    def gi_bias(b_ih, b_hh):
        return jnp.concatenate(
            [b_ih[:2 * h] + b_hh[:2 * h], b_ih[2 * h:]]).reshape(1, g3)

    bi0 = gi_bias(b_ih_0, b_hh_0)
    bi1 = gi_bias(b_ih_1, b_hh_1)
    bn0 = b_hh_0[2 * h:].reshape(1, h)
    bn1 = b_hh_1[2 * h:].reshape(1, h)

    call = pl.pallas_call(
        _fused_kernel,
        grid=(t // chunk,),
        in_specs=[
            pl.BlockSpec((rows, d), lambda c: (c, 0)),
            pl.BlockSpec((chunk, n, 1), lambda c: (c, 0, 0)),
            pl.BlockSpec((n, h), lambda c: (0, 0)),
            pl.BlockSpec((n, h), lambda c: (0, 0)),
            pl.BlockSpec((d, g3), lambda c: (0, 0)),
            pl.BlockSpec((1, g3), lambda c: (0, 0)),
            pl.BlockSpec((h, g3), lambda c: (0, 0)),
            pl.BlockSpec((1, h), lambda c: (0, 0)),
            pl.BlockSpec((h, g3), lambda c: (0, 0)),
            pl.BlockSpec((1, g3), lambda c: (0, 0)),
            pl.BlockSpec((h, g3), lambda c: (0, 0)),
            pl.BlockSpec((1, h), lambda c: (0, 0)),
        ],
        out_specs=[
            pl.BlockSpec((rows, h), lambda c: (c, 0)),
            pl.BlockSpec((n, h), lambda c: (0, 0)),
            pl.BlockSpec((n, h), lambda c: (0, 0)),
        ],
        out_shape=[
            jax.ShapeDtypeStruct((tn, h), jnp.float32),
            jax.ShapeDtypeStruct((n, h), jnp.float32),
            jax.ShapeDtypeStruct((n, h), jnp.float32),
        ],
        scratch_shapes=[
            pltpu.VMEM((n, h), jnp.float32),
            pltpu.VMEM((n, h), jnp.float32),
            pltpu.VMEM((rows, g3), jnp.float32),
            pltpu.VMEM((rows, g3), jnp.float32),
            pltpu.VMEM((rows, h), jnp.bfloat16),
        ],
        compiler_params=pltpu.CompilerParams(
            dimension_semantics=("arbitrary",),
        ),
    )

    args = (x, m3, hidden_states[0], hidden_states[1],
            W_ih_0.T.astype(jnp.bfloat16), bi0,
            W_hh_0.T.astype(jnp.bfloat16), bn0,
            W_ih_1.T.astype(jnp.bfloat16), bi1,
            W_hh_1.T.astype(jnp.bfloat16), bn1)

    out, h0n, h1n = call(*args)
    return out, jnp.stack([h0n, h1n], axis=0)
